# baseline (speedup 1.0000x reference)
import jax
import jax.numpy as jnp
from jax.experimental import pallas as pl
from jax.experimental.pallas import tpu as pltpu

_B, _S, _D = 4, 8192, 768
_RANK = 4
_E = 16
_SCALE = 1.0 / _RANK

_TS = 4096
_NS = _S // _TS


def _adapter_index(bb, ss, sid_ref):
    nxt = jnp.where(ss == _NS - 1, jnp.minimum(bb + 1, _B - 1), bb)
    return sid_ref[nxt]


def _fused_kernel(sid_ref, x_ref, Wt_ref, b_ref, A_ref, Bt_ref, out_ref, weff_ref):
    bb = pl.program_id(0)
    ss = pl.program_id(1)
    cur = jax.lax.rem(bb, 2)

    @pl.when((bb == 0) & (ss == 0))
    def _build_first():
        weff_ref[0] = Wt_ref[...] + _SCALE * jnp.dot(
            A_ref[0].T, Bt_ref[0], preferred_element_type=jnp.float32
        )

    out_ref[0] = (
        jnp.dot(x_ref[0], weff_ref[cur], preferred_element_type=jnp.float32)
        + b_ref[...]
    )

    @pl.when((ss == _NS - 1) & (bb < _B - 1))
    def _build_next():
        weff_ref[1 - cur] = Wt_ref[...] + _SCALE * jnp.dot(
            A_ref[0].T, Bt_ref[0], preferred_element_type=jnp.float32
        )


def kernel(x, subject_id, W, b, lora_A, lora_B):
    Wt = W.T
    Bt = lora_B.transpose(0, 2, 1)
    sid = subject_id.astype(jnp.int32)

    grid_spec = pltpu.PrefetchScalarGridSpec(
        num_scalar_prefetch=1,
        grid=(_B, _NS),
        in_specs=[
            pl.BlockSpec((1, _TS, _D), lambda bb, ss, sid_ref: (bb, ss, 0)),
            pl.BlockSpec((_D, _D), lambda bb, ss, sid_ref: (0, 0)),
            pl.BlockSpec((1, _D), lambda bb, ss, sid_ref: (0, 0)),
            pl.BlockSpec(
                (1, _RANK, _D),
                lambda bb, ss, sid_ref: (_adapter_index(bb, ss, sid_ref), 0, 0),
            ),
            pl.BlockSpec(
                (1, _RANK, _D),
                lambda bb, ss, sid_ref: (_adapter_index(bb, ss, sid_ref), 0, 0),
            ),
        ],
        out_specs=pl.BlockSpec((1, _TS, _D), lambda bb, ss, sid_ref: (bb, ss, 0)),
        scratch_shapes=[pltpu.VMEM((2, _D, _D), jnp.float32)],
    )

    return pl.pallas_call(
        _fused_kernel,
        grid_spec=grid_spec,
        out_shape=jax.ShapeDtypeStruct((_B, _S, _D), jnp.float32),
        compiler_params=pltpu.CompilerParams(
            dimension_semantics=("arbitrary", "arbitrary"),
            vmem_limit_bytes=100 * 1024 * 1024,
        ),
    )(sid, x, Wt, b.reshape(1, _D), lora_A, Bt)
